# baseline (speedup 1.0000x reference)
import functools

import jax
import jax.numpy as jnp
from jax import lax
from jax.experimental import pallas as pl
from jax.experimental.pallas import tpu as pltpu
from jax.experimental.pallas import tpu_sc as plsc

N = 10000
E = 320000
D = 128
NC = 2
NS = 16
NW = NC * NS
EPW = E // NW
K = 80
ITERS = EPW // K
NSTAGE = 5
IPS = ITERS // NSTAGE
NP = 10112
RPT = NP // NS


def _sc_gather_scatter(feat, ei):
    mesh = plsc.VectorSubcoreMesh(core_axis_name="c", subcore_axis_name="s")

    @functools.partial(
        pl.kernel,
        out_type=jax.ShapeDtypeStruct((NC, NP, D), jnp.float32),
        mesh=mesh,
        scratch_types=[
            pltpu.VMEM((IPS, K), jnp.int32),
            pltpu.VMEM((IPS, K), jnp.int32),
            pltpu.VMEM((K, D), jnp.float32),
            pltpu.VMEM((K, D), jnp.float32),
            pltpu.VMEM((K, D), jnp.float32),
            pltpu.VMEM((K, D), jnp.float32),
            pltpu.VMEM_SHARED((NP, D), jnp.float32),
            pltpu.SemaphoreType.DMA,
            pltpu.SemaphoreType.DMA,
            pltpu.SemaphoreType.DMA,
            pltpu.SemaphoreType.DMA,
            pltpu.SemaphoreType.DMA,
            pltpu.SemaphoreType.DMA,
            pltpu.SemaphoreType.DMA,
            pltpu.SemaphoreType.DMA,
            pltpu.SemaphoreType.DMA,
            pltpu.SemaphoreType.DMA,
        ],
    )
    def k(feat_hbm, ei_hbm, out_hbm, sx, dx, rows0, rows1, rows2, rows3,
          accum, g0, g1, g2, g3, s0, s1, s2, s3, isem, zsem):
        c = lax.axis_index("c")
        s = lax.axis_index("s")
        wid = c * NS + s
        rows = (rows0, rows1, rows2, rows3)
        gsem = (g0, g1, g2, g3)
        ssem = (s0, s1, s2, s3)

        pltpu.async_copy(ei_hbm.at[0, wid, 0], sx, isem)
        pltpu.async_copy(ei_hbm.at[1, wid, 0], dx, isem)

        z = jnp.zeros((16,), jnp.float32)

        def zrow(i, _):
            for j in range(D // 16):
                rows0[i, pl.ds(j * 16, 16)] = z
            return 0

        lax.fori_loop(0, K, zrow, 0)
        r0 = s * RPT
        zcuts = list(range(0, RPT - K + 1, K)) + [RPT - K]
        for zo in zcuts:
            pltpu.async_copy(rows0, accum.at[pl.ds(r0 + zo, K)], zsem)
        for _ in zcuts:
            pltpu.make_async_copy(rows0, accum.at[pl.ds(r0, K)], zsem).wait()
        plsc.subcore_barrier()

        def gat(sx, ch, b):
            return pltpu.async_copy(feat_hbm.at[sx.at[ch]], rows[b], gsem[b])

        def gat_w(sx, ch, b):
            pltpu.make_async_copy(feat_hbm.at[sx.at[ch]], rows[b], gsem[b]).wait()

        def sca(dx, ch, b):
            return pltpu.async_copy(rows[b], accum.at[dx.at[ch]], ssem[b],
                                    add=True)

        def sca_w(dx, ch, b):
            pltpu.make_async_copy(rows[b], accum.at[dx.at[ch]], ssem[b]).wait()

        for sg in range(NSTAGE):
            if sg > 0:
                pltpu.async_copy(ei_hbm.at[0, wid, sg], sx, isem)
                pltpu.async_copy(ei_hbm.at[1, wid, sg], dx, isem)
            pltpu.make_async_copy(ei_hbm.at[0, wid, sg], sx, isem).wait()
            pltpu.make_async_copy(ei_hbm.at[1, wid, sg], dx, isem).wait()

            def slot(ch, phase, first, last):
                b = phase % 4
                b2 = (phase + 2) % 4
                gat_w(sx, ch, b)
                sca(dx, ch, b)
                if not first:
                    sca_w(dx, ch - 2, b2)
                if not last:
                    gat(sx, ch + 2, b2)

            gat(sx, 0, 0)
            gat(sx, 1, 1)
            slot(0, 0, True, False)
            slot(1, 1, True, False)

            def body(i, _):
                c0 = 2 + 4 * i
                for j in range(4):
                    slot(c0 + j, 2 + j, False, False)
                return 0

            nbody = (IPS - 6) // 4
            lax.fori_loop(0, nbody, body, 0)
            for ch in range(2 + 4 * nbody, IPS):
                slot(ch, ch, False, ch + 2 >= IPS)
            sca_w(dx, IPS - 2, (IPS - 2) % 4)
            sca_w(dx, IPS - 1, (IPS - 1) % 4)

        plsc.subcore_barrier()

        pltpu.sync_copy(accum.at[pl.ds(r0, RPT)], out_hbm.at[c, pl.ds(r0, RPT)])

    return k(feat, ei)


def _tc_final(partials, feat, w_neigh, w_self, b_neigh):
    bn = 1000

    def body(p_ref, f_ref, wn_ref, ws_ref, b_ref, o_ref):
        agg = p_ref[0] + p_ref[1]
        o_ref[...] = (
            jnp.dot(agg, wn_ref[...], preferred_element_type=jnp.float32)
            + jnp.dot(f_ref[...], ws_ref[...], preferred_element_type=jnp.float32)
            + b_ref[...]
        )

    return pl.pallas_call(
        body,
        grid=(N // bn,),
        in_specs=[
            pl.BlockSpec((NC, bn, D), lambda i: (0, i, 0)),
            pl.BlockSpec((bn, D), lambda i: (i, 0)),
            pl.BlockSpec((D, D), lambda i: (0, 0)),
            pl.BlockSpec((D, D), lambda i: (0, 0)),
            pl.BlockSpec((1, D), lambda i: (0, 0)),
        ],
        out_specs=pl.BlockSpec((bn, D), lambda i: (i, 0)),
        out_shape=jax.ShapeDtypeStruct((N, D), jnp.float32),
    )(partials, feat, w_neigh, w_self, b_neigh.reshape(1, D))


def kernel(feat, edge_index, W_neigh, b_neigh, W_self):
    ei = edge_index.reshape(2, NW, NSTAGE, IPS, K)
    partials = _sc_gather_scatter(feat, ei)
    return _tc_final(partials, feat, W_neigh, W_self, b_neigh)


# speedup vs baseline: 1.1414x; 1.1414x over previous
import functools

import jax
import jax.numpy as jnp
from jax import lax
from jax.experimental import pallas as pl
from jax.experimental.pallas import tpu as pltpu
from jax.experimental.pallas import tpu_sc as plsc

N = 10000
E = 320000
D = 128
NC = 2
NS = 16
NW = NC * NS
EPW = E // NW
K = 80
ITERS = EPW // K
NSTAGE = 5
IPS = ITERS // NSTAGE
NP = 10240
RPT = NP // NS


def _sc_gather_scatter(feat, ei):
    mesh = plsc.VectorSubcoreMesh(core_axis_name="c", subcore_axis_name="s")

    @functools.partial(
        pl.kernel,
        out_type=jax.ShapeDtypeStruct((NC, NP, D), jnp.float32),
        mesh=mesh,
        scratch_types=[
            pltpu.VMEM((IPS, K), jnp.int32),
            pltpu.VMEM((IPS, K), jnp.int32),
            pltpu.VMEM((IPS, K), jnp.int32),
            pltpu.VMEM((IPS, K), jnp.int32),
            pltpu.VMEM((K, D), jnp.float32),
            pltpu.VMEM((K, D), jnp.float32),
            pltpu.VMEM((K, D), jnp.float32),
            pltpu.VMEM_SHARED((NP, D), jnp.float32),
            pltpu.SemaphoreType.DMA,
            pltpu.SemaphoreType.DMA,
            pltpu.SemaphoreType.DMA,
            pltpu.SemaphoreType.DMA,
            pltpu.SemaphoreType.DMA,
            pltpu.SemaphoreType.DMA,
            pltpu.SemaphoreType.DMA,
            pltpu.SemaphoreType.DMA,
            pltpu.SemaphoreType.DMA,
        ],
    )
    def k(feat_hbm, ei_hbm, out_hbm, sixa, sixb, dixa, dixb, rows0, rows1,
          rows2, accum, g0, g1, g2, s0, s1, s2, ia, ib, zsem):
        c = lax.axis_index("c")
        s = lax.axis_index("s")
        wid = c * NS + s
        six = (sixa, sixb)
        dix = (dixa, dixb)
        isem = (ia, ib)
        rows = (rows0, rows1, rows2)
        gsem = (g0, g1, g2)
        ssem = (s0, s1, s2)

        pltpu.async_copy(ei_hbm.at[0, wid, 0], six[0], isem[0])
        pltpu.async_copy(ei_hbm.at[1, wid, 0], dix[0], isem[0])

        z = jnp.zeros((16,), jnp.float32)

        def zrow(i, _):
            for j in range(D // 16):
                rows0[i, pl.ds(j * 16, 16)] = z
            return 0

        lax.fori_loop(0, K, zrow, 0)
        r0 = s * RPT
        for j in range(RPT // K):
            pltpu.async_copy(rows0, accum.at[pl.ds(r0 + j * K, K)], zsem)
        for j in range(RPT // K):
            pltpu.make_async_copy(rows0, accum.at[pl.ds(r0, K)], zsem).wait()
        plsc.subcore_barrier()

        def gat(sx, ch, b):
            return pltpu.async_copy(feat_hbm.at[sx.at[ch]], rows[b], gsem[b])

        def gat_w(sx, ch, b):
            pltpu.make_async_copy(feat_hbm.at[sx.at[ch]], rows[b], gsem[b]).wait()

        def sca(dx, ch, b):
            return pltpu.async_copy(rows[b], accum.at[dx.at[ch]], ssem[b],
                                    add=True)

        def sca_w(dx, ch, b):
            pltpu.make_async_copy(rows[b], accum.at[dx.at[ch]], ssem[b]).wait()

        for sg in range(NSTAGE):
            p = sg % 2
            sx, dx = six[p], dix[p]
            pltpu.make_async_copy(ei_hbm.at[0, wid, sg], sx, isem[p]).wait()
            pltpu.make_async_copy(ei_hbm.at[1, wid, sg], dx, isem[p]).wait()
            if sg + 1 < NSTAGE:
                q = (sg + 1) % 2
                pltpu.async_copy(ei_hbm.at[0, wid, sg + 1], six[q], isem[q])
                pltpu.async_copy(ei_hbm.at[1, wid, sg + 1], dix[q], isem[q])

            gat(sx, 0, 0)
            gat(sx, 1, 1)
            gat_w(sx, 0, 0)
            sca(dx, 0, 0)
            gat(sx, 2, 2)
            gat_w(sx, 1, 1)
            sca(dx, 1, 1)
            sca_w(dx, 0, 0)
            gat(sx, 3, 0)
            gat_w(sx, 2, 2)
            sca(dx, 2, 2)
            sca_w(dx, 1, 1)
            gat(sx, 4, 1)

            def body(i, _):
                c0 = 3 + 3 * i
                gat_w(sx, c0, 0)
                sca(dx, c0, 0)
                sca_w(dx, c0 - 1, 2)
                gat(sx, c0 + 2, 2)
                gat_w(sx, c0 + 1, 1)
                sca(dx, c0 + 1, 1)
                sca_w(dx, c0, 0)
                gat(sx, c0 + 3, 0)
                gat_w(sx, c0 + 2, 2)
                sca(dx, c0 + 2, 2)
                sca_w(dx, c0 + 1, 1)

                @pl.when(c0 + 4 < IPS)
                def _():
                    gat(sx, c0 + 4, 1)

                return 0

            lax.fori_loop(0, (IPS - 4) // 3, body, 0)
            gat_w(sx, IPS - 1, 0)
            sca(dx, IPS - 1, 0)
            sca_w(dx, IPS - 2, 2)
            sca_w(dx, IPS - 1, 0)

        plsc.subcore_barrier()

        pltpu.sync_copy(accum.at[pl.ds(r0, RPT)], out_hbm.at[c, pl.ds(r0, RPT)])

    return k(feat, ei)


def _tc_base(feat, w_self, b_neigh):
    bn = 2000

    def body(f_ref, ws_ref, b_ref, o_ref):
        o_ref[...] = (
            jnp.dot(f_ref[...], ws_ref[...], preferred_element_type=jnp.float32)
            + b_ref[...]
        )

    return pl.pallas_call(
        body,
        grid=(N // bn,),
        in_specs=[
            pl.BlockSpec((bn, D), lambda i: (i, 0)),
            pl.BlockSpec((D, D), lambda i: (0, 0)),
            pl.BlockSpec((1, D), lambda i: (0, 0)),
        ],
        out_specs=pl.BlockSpec((bn, D), lambda i: (i, 0)),
        out_shape=jax.ShapeDtypeStruct((N, D), jnp.float32),
    )(feat, w_self, b_neigh.reshape(1, D))


def _tc_final(partials, base, w_neigh):
    bn = 2000

    def body(p_ref, b_ref, wn_ref, o_ref):
        agg = p_ref[0] + p_ref[1]
        o_ref[...] = (
            jnp.dot(agg, wn_ref[...], preferred_element_type=jnp.float32)
            + b_ref[...]
        )

    return pl.pallas_call(
        body,
        grid=(N // bn,),
        in_specs=[
            pl.BlockSpec((NC, bn, D), lambda i: (0, i, 0)),
            pl.BlockSpec((bn, D), lambda i: (i, 0)),
            pl.BlockSpec((D, D), lambda i: (0, 0)),
        ],
        out_specs=pl.BlockSpec((bn, D), lambda i: (i, 0)),
        out_shape=jax.ShapeDtypeStruct((N, D), jnp.float32),
    )(partials, base, w_neigh)


def kernel(feat, edge_index, W_neigh, b_neigh, W_self):
    ei = edge_index.reshape(2, NW, NSTAGE, IPS, K)
    partials = _sc_gather_scatter(feat, ei)
    base = _tc_base(feat, W_self, b_neigh)
    return _tc_final(partials, base, W_neigh)
